# baseline (speedup 1.0000x reference)
import functools

import jax
import jax.numpy as jnp
from jax import lax
from jax.experimental import pallas as pl
from jax.experimental.pallas import tpu as pltpu
from jax.experimental.pallas import tpu_sc as plsc

NC = 2
NS = 16
NW = NC * NS
L = 16

R = 128
C = 100000
CP = 100096
HW = CP // 2
N = 16384
ROWS_PER_W = R // NW
SRC_CHUNK = 4096
NSC = N // SRC_CHUNK
ZGROUPS = HW // L
ZUNROLL = 23
SUNROLL = 4


def _delta_body(idx_hbm, src_hbm, delta_hbm, idx_v, buf0, buf1,
                sv0, sv1, sems, ssems):
    bufs = [buf0, buf1]
    svs = [sv0, sv1]
    wid = lax.axis_index("s") * NC + lax.axis_index("c")
    pltpu.sync_copy(idx_hbm, idx_v)
    zeros = jnp.zeros((L,), jnp.float32)
    out_h = [None] * (2 * ROWS_PER_W)
    for p in range(2 * ROWS_PER_W):
        b = p % 2
        h = p // ROWS_PER_W
        r = wid * ROWS_PER_W + (p % ROWS_PER_W)
        lo = h * HW
        if p >= 2:
            out_h[p - 2].wait()

        src_h = [None] * NSC
        src_h[0] = pltpu.async_copy(
            src_hbm.at[r, pl.ds(0, SRC_CHUNK)], svs[0], ssems.at[0])

        def zbody(i, _, b=b):
            for u in range(ZUNROLL):
                bufs[b][pl.ds((i * ZUNROLL + u) * L, L)] = zeros
            return 0

        lax.fori_loop(0, ZGROUPS // ZUNROLL, zbody, 0)

        for ch in range(NSC):
            sb = ch % 2
            src_h[ch].wait()
            if ch + 1 < NSC:
                src_h[ch + 1] = pltpu.async_copy(
                    src_hbm.at[r, pl.ds((ch + 1) * SRC_CHUNK, SRC_CHUNK)],
                    svs[1 - sb], ssems.at[1 - sb])

            def sbody(i, _, b=b, sb=sb, ch=ch, lo=lo):
                for u in range(SUNROLL):
                    g = i * SUNROLL + u
                    idxs = idx_v[pl.ds(ch * SRC_CHUNK + g * L, L)]
                    vals = svs[sb][pl.ds(g * L, L)]
                    cols = idxs - lo
                    cols = jnp.where((cols >= 0) & (cols < HW), cols, HW)
                    plsc.addupdate_scatter(bufs[b], [cols], vals)
                return 0

            lax.fori_loop(0, SRC_CHUNK // L // SUNROLL, sbody, 0)

        out_h[p] = pltpu.async_copy(bufs[b].at[pl.ds(0, HW)],
                                    delta_hbm.at[h, r], sems.at[b])
    out_h[2 * ROWS_PER_W - 2].wait()
    out_h[2 * ROWS_PER_W - 1].wait()


def _delta(idx32, src):
    mesh = plsc.VectorSubcoreMesh(core_axis_name="c", subcore_axis_name="s")
    f = pl.kernel(
        _delta_body,
        out_type=jax.ShapeDtypeStruct((2, R, HW), jnp.float32),
        mesh=mesh,
        scratch_types=[
            pltpu.VMEM((N,), jnp.int32),
            pltpu.VMEM((HW + L,), jnp.float32),
            pltpu.VMEM((HW + L,), jnp.float32),
            pltpu.VMEM((SRC_CHUNK,), jnp.float32),
            pltpu.VMEM((SRC_CHUNK,), jnp.float32),
            pltpu.SemaphoreType.DMA((2,)),
            pltpu.SemaphoreType.DMA((2,)),
        ],
        compiler_params=pltpu.CompilerParams(needs_layout_passes=False),
    )
    return f(idx32, src)


BW = 2176
NB = HW // BW


def _add_body(x_ref, d_ref, o_ref):
    o_ref[...] = x_ref[...] + d_ref[0]


def _apply(x, delta):
    return pl.pallas_call(
        _add_body,
        out_shape=jax.ShapeDtypeStruct((R, C), jnp.float32),
        grid=(2, NB),
        in_specs=[
            pl.BlockSpec((R, BW), lambda h, i: (0, h * NB + i)),
            pl.BlockSpec((1, R, BW), lambda h, i: (h, 0, i)),
        ],
        out_specs=pl.BlockSpec((R, BW), lambda h, i: (0, h * NB + i)),
    )(x, delta)


def kernel(x, indices, src):
    idx32 = indices.astype(jnp.int32)
    return _apply(x, _delta(idx32, src))


# speedup vs baseline: 1.4685x; 1.4685x over previous
import jax
import jax.numpy as jnp
from jax import lax
from jax.experimental import pallas as pl
from jax.experimental.pallas import tpu as pltpu
from jax.experimental.pallas import tpu_sc as plsc

NC = 2
NS = 16
NW = NC * NS
L = 16

R = 128
C = 100000
N = 16384
ROWS_PER_W = R // NW
SRC_CHUNK = 4096
NSC = N // SRC_CHUNK
SUNROLL = 4


def _scatter_body(x_hbm, idx_hbm, src_hbm, out_hbm, idx_v, row_v,
                  sv0, sv1, ssems):
    svs = [sv0, sv1]
    wid = lax.axis_index("s") * NC + lax.axis_index("c")
    pltpu.sync_copy(idx_hbm, idx_v)
    for rr in range(ROWS_PER_W):
        r = wid * ROWS_PER_W + rr
        src_h = [None] * NSC
        src_h[0] = pltpu.async_copy(
            src_hbm.at[r, pl.ds(0, SRC_CHUNK)], svs[0], ssems.at[0])
        pltpu.sync_copy(x_hbm.at[r], row_v)
        for ch in range(NSC):
            sb = ch % 2
            src_h[ch].wait()
            if ch + 1 < NSC:
                src_h[ch + 1] = pltpu.async_copy(
                    src_hbm.at[r, pl.ds((ch + 1) * SRC_CHUNK, SRC_CHUNK)],
                    svs[1 - sb], ssems.at[1 - sb])

            def sbody(i, _, sb=sb, ch=ch):
                for u in range(SUNROLL):
                    g = i * SUNROLL + u
                    idxs = idx_v[pl.ds(ch * SRC_CHUNK + g * L, L)]
                    vals = svs[sb][pl.ds(g * L, L)]
                    plsc.addupdate_scatter(row_v, [idxs], vals)
                return 0

            lax.fori_loop(0, SRC_CHUNK // L // SUNROLL, sbody, 0)
        pltpu.sync_copy(row_v, out_hbm.at[r])


def kernel(x, indices, src):
    idx32 = indices.astype(jnp.int32)
    mesh = plsc.VectorSubcoreMesh(core_axis_name="c", subcore_axis_name="s")
    f = pl.kernel(
        _scatter_body,
        out_type=jax.ShapeDtypeStruct((R, C), jnp.float32),
        mesh=mesh,
        scratch_types=[
            pltpu.VMEM((N,), jnp.int32),
            pltpu.VMEM((C,), jnp.float32),
            pltpu.VMEM((SRC_CHUNK,), jnp.float32),
            pltpu.VMEM((SRC_CHUNK,), jnp.float32),
            pltpu.SemaphoreType.DMA((2,)),
        ],
        compiler_params=pltpu.CompilerParams(needs_layout_passes=False),
    )
    return f(x, idx32, src)


# speedup vs baseline: 1.4766x; 1.0055x over previous
import jax
import jax.numpy as jnp
from jax import lax
from jax.experimental import pallas as pl
from jax.experimental.pallas import tpu as pltpu
from jax.experimental.pallas import tpu_sc as plsc

NC = 2
NS = 16
NW = NC * NS
L = 16

R = 128
C = 100000
N = 16384
ROWS_PER_W = R // NW
SRC_CHUNK = 4096
NSC = N // SRC_CHUNK
SUNROLL = 8


def _scatter_body(x_hbm, idx_hbm, src_hbm, out_hbm, idx_v, row_v,
                  sv0, sv1, ssems, isem):
    svs = [sv0, sv1]
    wid = lax.axis_index("s") * NC + lax.axis_index("c")
    idx_h = pltpu.async_copy(idx_hbm, idx_v, isem)
    idx_waited = [False]
    for rr in range(ROWS_PER_W):
        r = wid * ROWS_PER_W + rr
        src_h = [None] * NSC
        src_h[0] = pltpu.async_copy(
            src_hbm.at[r, pl.ds(0, SRC_CHUNK)], svs[0], ssems.at[0])
        pltpu.sync_copy(x_hbm.at[r], row_v)
        if not idx_waited[0]:
            idx_h.wait()
            idx_waited[0] = True
        for ch in range(NSC):
            sb = ch % 2
            src_h[ch].wait()
            if ch + 1 < NSC:
                src_h[ch + 1] = pltpu.async_copy(
                    src_hbm.at[r, pl.ds((ch + 1) * SRC_CHUNK, SRC_CHUNK)],
                    svs[1 - sb], ssems.at[1 - sb])

            def sbody(i, _, sb=sb, ch=ch):
                for u in range(SUNROLL):
                    g = i * SUNROLL + u
                    idxs = idx_v[pl.ds(ch * SRC_CHUNK + g * L, L)]
                    vals = svs[sb][pl.ds(g * L, L)]
                    plsc.addupdate_scatter(row_v, [idxs], vals)
                return 0

            lax.fori_loop(0, SRC_CHUNK // L // SUNROLL, sbody, 0)
        pltpu.sync_copy(row_v, out_hbm.at[r])


def kernel(x, indices, src):
    idx32 = indices.astype(jnp.int32)
    mesh = plsc.VectorSubcoreMesh(core_axis_name="c", subcore_axis_name="s")
    f = pl.kernel(
        _scatter_body,
        out_type=jax.ShapeDtypeStruct((R, C), jnp.float32),
        mesh=mesh,
        scratch_types=[
            pltpu.VMEM((N,), jnp.int32),
            pltpu.VMEM((C,), jnp.float32),
            pltpu.VMEM((SRC_CHUNK,), jnp.float32),
            pltpu.VMEM((SRC_CHUNK,), jnp.float32),
            pltpu.SemaphoreType.DMA((2,)),
            pltpu.SemaphoreType.DMA,
        ],
        compiler_params=pltpu.CompilerParams(needs_layout_passes=False),
    )
    return f(x, idx32, src)
